# baseline (speedup 1.0000x reference)
import functools

import jax
import jax.numpy as jnp
from jax import lax
from jax.experimental import pallas as pl
from jax.experimental.pallas import tpu as pltpu
from jax.experimental.pallas import tpu_sc as plsc

_BATCH = 4096
_NEG = 200
_NTAIL = _NEG + 1
_ENT_DIM = 128
_FS_DIM = 64
_REL_DIM = 64
_OREL_DIM = 128

_NW = 32
_BW = _BATCH // _NW
_NBUF = 3
_NGRP = _NTAIL // _NBUF
_L = 16
_KB = _BW // _L


def _sc_body(tail_idx, pos0, pos1, ent_tab, orel_tab, rel_tab, fs2,
             tail_out, head_out, rel_out, orel_out,
             idx_all, buf0, buf1, buf2, qb0, qb1, qb2, sl0, sl1, sl2,
             pos_v, wide_v, sem, gsem, wsem):
    c = lax.axis_index("c")
    s = lax.axis_index("s")
    wid = s * 2 + c
    b0 = wid * _BW
    bufs = (buf0, buf1, buf2)
    qbs = (qb0, qb1, qb2)
    slabs = (sl0, sl1, sl2)
    iotas = [lax.broadcasted_iota(jnp.int32, (_L,), 0) + kb * _L
             for kb in range(_KB)]

    def fill_q(j, qb):
        for kb in range(_KB):
            qb[pl.ds(kb * _L, _L)] = (
                idx_all[j, pl.ds(kb * _L, _L)] >> 1)

    pltpu.sync_copy(tail_idx.at[wid], idx_all)

    for bb in range(_NBUF):
        fill_q(bb, qbs[bb])
        pltpu.async_copy(fs2.at[qbs[bb]], bufs[bb], gsem.at[bb])

    pltpu.sync_copy(pos0.at[wid], pos_v)
    pltpu.async_copy(ent_tab.at[pos_v], wide_v, sem).wait()
    pltpu.sync_copy(wide_v, head_out.at[pl.ds(b0, _BW)])
    pltpu.sync_copy(pos1.at[wid], pos_v)
    pltpu.async_copy(orel_tab.at[pos_v], wide_v, sem).wait()
    pltpu.sync_copy(wide_v, orel_out.at[pl.ds(b0, _BW)])
    pltpu.async_copy(rel_tab.at[pos_v], wide_v, sem).wait()
    pltpu.sync_copy(wide_v, rel_out.at[pl.ds(b0, _BW)])

    def turn(g, carry):
        for bb in range(_NBUF):
            j = _NBUF * g + bb
            buf, qb, slab = bufs[bb], qbs[bb], slabs[bb]
            pltpu.make_async_copy(fs2.at[qb], buf, gsem.at[bb]).wait()

            @pl.when(g >= 1)
            def _():
                pltpu.make_async_copy(
                    slab, tail_out.at[0, :, pl.ds(b0, _BW)],
                    wsem.at[bb]).wait()

            for kb in range(_KB):
                hv = (idx_all[j, pl.ds(kb * _L, _L)] & 1) * _FS_DIM
                for d in range(_FS_DIM):
                    v = plsc.load_gather(buf, [iotas[kb], hv + d])
                    slab[d, pl.ds(kb * _L, _L)] = v

            pltpu.async_copy(slab, tail_out.at[j, :, pl.ds(b0, _BW)],
                             wsem.at[bb])

            @pl.when(j + _NBUF < _NTAIL)
            def _():
                fill_q(j + _NBUF, qb)
                pltpu.async_copy(fs2.at[qb], buf, gsem.at[bb])

        return carry

    lax.fori_loop(0, _NGRP, turn, 0)

    for bb in range(_NBUF):
        pltpu.make_async_copy(
            slabs[bb], tail_out.at[0, :, pl.ds(b0, _BW)], wsem.at[bb]).wait()


@jax.jit
def _gather_all(tail_idx, pos0, pos1, ent_tab, orel_tab, rel_tab, fs2):
    mesh = plsc.VectorSubcoreMesh(core_axis_name="c", subcore_axis_name="s")
    f = functools.partial(
        pl.kernel,
        mesh=mesh,
        compiler_params=pltpu.CompilerParams(needs_layout_passes=False),
        out_type=[
            jax.ShapeDtypeStruct((_NTAIL, _FS_DIM, _BATCH), jnp.float32),
            jax.ShapeDtypeStruct((_BATCH, _ENT_DIM), jnp.float32),
            jax.ShapeDtypeStruct((_BATCH, 2 * _REL_DIM), jnp.float32),
            jax.ShapeDtypeStruct((_BATCH, _OREL_DIM), jnp.float32),
        ],
        scratch_types=[
            pltpu.VMEM((_NTAIL, _BW), jnp.int32),
            pltpu.VMEM((_BW, 2 * _FS_DIM), jnp.float32),
            pltpu.VMEM((_BW, 2 * _FS_DIM), jnp.float32),
            pltpu.VMEM((_BW, 2 * _FS_DIM), jnp.float32),
            pltpu.VMEM((_BW,), jnp.int32),
            pltpu.VMEM((_BW,), jnp.int32),
            pltpu.VMEM((_BW,), jnp.int32),
            pltpu.VMEM((_FS_DIM, _BW), jnp.float32),
            pltpu.VMEM((_FS_DIM, _BW), jnp.float32),
            pltpu.VMEM((_FS_DIM, _BW), jnp.float32),
            pltpu.VMEM((_BW,), jnp.int32),
            pltpu.VMEM((_BW, _ENT_DIM), jnp.float32),
            pltpu.SemaphoreType.DMA,
            pltpu.SemaphoreType.DMA((_NBUF,)),
            pltpu.SemaphoreType.DMA((_NBUF,)),
        ],
    )(_sc_body)
    return f(tail_idx, pos0, pos1, ent_tab, orel_tab, rel_tab, fs2)


def kernel(positive, negative, entity_embedding, origin_relation_embedding,
           relation_embedding, entity_embedding_FS):
    positive = positive.astype(jnp.int32)
    negative = negative.astype(jnp.int32)
    tail_idx = jnp.concatenate([positive[:, 2:3], negative], axis=1)
    tail_idx = tail_idx.reshape(_NW, _BW, _NTAIL).transpose(0, 2, 1)
    pos0 = positive[:, 0].reshape(_NW, _BW)
    pos1 = positive[:, 1].reshape(_NW, _BW)
    fs2 = entity_embedding_FS.reshape(-1, 2 * _FS_DIM)
    rel_pad = jnp.pad(relation_embedding, ((0, 0), (0, _REL_DIM)))
    tail_t, head, rel_p, orel = _gather_all(
        tail_idx, pos0, pos1, entity_embedding, origin_relation_embedding,
        rel_pad, fs2)
    tail = tail_t.transpose(2, 0, 1)
    return (head.reshape(_BATCH, 1, _ENT_DIM),
            rel_p[:, :_REL_DIM].reshape(_BATCH, 1, _REL_DIM),
            tail,
            orel.reshape(_BATCH, 1, _OREL_DIM))


# speedup vs baseline: 1.0497x; 1.0497x over previous
import functools

import jax
import jax.numpy as jnp
from jax import lax
from jax.experimental import pallas as pl
from jax.experimental.pallas import tpu as pltpu
from jax.experimental.pallas import tpu_sc as plsc

_BATCH = 4096
_NEG = 200
_NENT = 1000000
_ENT_DIM = 128
_FS_DIM = 64
_REL_DIM = 64
_OREL_DIM = 128

_NW = 32
_TAIL_ROWS = _BATCH * (_NEG + 1)
_PER_W = _TAIL_ROWS // _NW
_SUBW = 128
_NCHUNK = _PER_W // _SUBW
_NBUF = 6
_POS_W = _BATCH // _NW
_TPC = 512

assert _PER_W * _NW == _TAIL_ROWS
assert _NCHUNK * _SUBW == _PER_W


def _transpose_pad_body(x_ref, o_ref):
    o_ref[:, 0:_FS_DIM] = x_ref[...].T


@jax.jit
def _transpose_pad(fs_t):
    grid = (pl.cdiv(_NENT, _TPC),)
    return pl.pallas_call(
        _transpose_pad_body,
        grid=grid,
        in_specs=[pl.BlockSpec((_FS_DIM, _TPC), lambda i: (0, i))],
        out_specs=pl.BlockSpec((_TPC, 2 * _FS_DIM), lambda i: (i, 0)),
        out_shape=jax.ShapeDtypeStruct((_NENT, 2 * _FS_DIM), jnp.float32),
        compiler_params=pltpu.CompilerParams(
            dimension_semantics=("arbitrary",)),
    )(fs_t)


def _sc_body(tail_idx, pos0, pos1, ent_tab, orel_tab, rel_tab, fs_tab,
             tail_out, head_out, rel_out, orel_out,
             idx_all, bufs, pos_v, wide_v, rel_v, sem, gsem, wsem):
    c = lax.axis_index("c")
    s = lax.axis_index("s")
    wid = s * 2 + c

    pltpu.sync_copy(tail_idx.at[wid], idx_all)

    for b in range(_NBUF):
        pltpu.async_copy(fs_tab.at[idx_all.at[b]], bufs.at[b], gsem.at[b])

    pltpu.sync_copy(pos0.at[wid], pos_v)
    pltpu.async_copy(ent_tab.at[pos_v], wide_v, sem).wait()
    pltpu.sync_copy(wide_v, head_out.at[wid])
    pltpu.sync_copy(pos1.at[wid], pos_v)
    pltpu.async_copy(orel_tab.at[pos_v], wide_v, sem).wait()
    pltpu.sync_copy(wide_v, orel_out.at[wid])
    pltpu.async_copy(rel_tab.at[pos_v], rel_v, sem).wait()
    pltpu.sync_copy(rel_v, rel_out.at[wid])

    def step(i, carry):
        b = lax.rem(i, _NBUF)
        pltpu.make_async_copy(
            fs_tab.at[idx_all.at[i]], bufs.at[b], gsem.at[b]).wait()
        pltpu.async_copy(bufs.at[b], tail_out.at[wid, i], wsem.at[b])

        ip = i - 1
        nxt = ip + _NBUF

        @pl.when(jnp.logical_and(ip >= 0, nxt < _NCHUNK))
        def _():
            bp = lax.rem(ip, _NBUF)
            pltpu.make_async_copy(
                bufs.at[bp], tail_out.at[wid, ip], wsem.at[bp]).wait()
            pltpu.async_copy(
                fs_tab.at[idx_all.at[nxt]], bufs.at[bp], gsem.at[bp])

        return carry

    lax.fori_loop(0, _NCHUNK, step, 0)

    for b in range(_NBUF):
        pltpu.make_async_copy(bufs.at[b], tail_out.at[wid, 0], wsem.at[b]).wait()


@jax.jit
def _gather_all(tail_idx, pos0, pos1, ent_tab, orel_tab, rel_tab, fs_tab):
    mesh = plsc.VectorSubcoreMesh(core_axis_name="c", subcore_axis_name="s")
    f = functools.partial(
        pl.kernel,
        mesh=mesh,
        compiler_params=pltpu.CompilerParams(use_tc_tiling_on_sc=False),
        out_type=[
            jax.ShapeDtypeStruct((_NW, _NCHUNK, _SUBW, _FS_DIM), jnp.float32),
            jax.ShapeDtypeStruct((_NW, _POS_W, _ENT_DIM), jnp.float32),
            jax.ShapeDtypeStruct((_NW, _POS_W, _REL_DIM), jnp.float32),
            jax.ShapeDtypeStruct((_NW, _POS_W, _OREL_DIM), jnp.float32),
        ],
        scratch_types=[
            pltpu.VMEM((_NCHUNK, _SUBW), jnp.int32),
            pltpu.VMEM((_NBUF, _SUBW, _FS_DIM), jnp.float32),
            pltpu.VMEM((_POS_W,), jnp.int32),
            pltpu.VMEM((_POS_W, _ENT_DIM), jnp.float32),
            pltpu.VMEM((_POS_W, _REL_DIM), jnp.float32),
            pltpu.SemaphoreType.DMA,
            pltpu.SemaphoreType.DMA((_NBUF,)),
            pltpu.SemaphoreType.DMA((_NBUF,)),
        ],
    )(_sc_body)
    return f(tail_idx, pos0, pos1, ent_tab, orel_tab, rel_tab, fs_tab)


def kernel(positive, negative, entity_embedding, origin_relation_embedding,
           relation_embedding, entity_embedding_FS):
    positive = positive.astype(jnp.int32)
    negative = negative.astype(jnp.int32)
    tail_idx = jnp.concatenate([positive[:, 2:3], negative], axis=1) * 2
    tail_idx = tail_idx.reshape(_NW, _NCHUNK, _SUBW)
    pos0 = positive[:, 0].reshape(_NW, _POS_W)
    pos1 = positive[:, 1].reshape(_NW, _POS_W)
    fs_rows = _transpose_pad(entity_embedding_FS.T)
    fs_v = fs_rows.reshape(2 * _NENT, _FS_DIM)
    tail, head, rel, orel = _gather_all(
        tail_idx, pos0, pos1, entity_embedding, origin_relation_embedding,
        relation_embedding, fs_v)
    return (head.reshape(_BATCH, 1, _ENT_DIM),
            rel.reshape(_BATCH, 1, _REL_DIM),
            tail.reshape(_BATCH, _NEG + 1, _FS_DIM),
            orel.reshape(_BATCH, 1, _OREL_DIM))


# speedup vs baseline: 1.6651x; 1.5862x over previous
import functools

import jax
import jax.numpy as jnp
from jax import lax
from jax.experimental import pallas as pl
from jax.experimental.pallas import tpu as pltpu
from jax.experimental.pallas import tpu_sc as plsc

_BATCH = 4096
_NEG = 200
_NENT = 1000000
_ENT_DIM = 128
_FS_DIM = 64
_REL_DIM = 64
_OREL_DIM = 128

_NW = 32
_TAIL_ROWS = _BATCH * (_NEG + 1)
_PER_W = _TAIL_ROWS // _NW
_SUBW = 128
_NCHUNK = _PER_W // _SUBW
_NBUF = 6
_POS_W = _BATCH // _NW
_TPC = 2048

assert _PER_W * _NW == _TAIL_ROWS
assert _NCHUNK * _SUBW == _PER_W


def _transpose_pad_body(x_ref, o_ref):
    xt = x_ref[...].T
    o_ref[...] = jnp.concatenate([xt, xt], axis=1)


@jax.jit
def _transpose_pad(fs_t):
    grid = (pl.cdiv(_NENT, _TPC),)
    return pl.pallas_call(
        _transpose_pad_body,
        grid=grid,
        in_specs=[pl.BlockSpec((_FS_DIM, _TPC), lambda i: (0, i))],
        out_specs=pl.BlockSpec((_TPC, 2 * _FS_DIM), lambda i: (i, 0)),
        out_shape=jax.ShapeDtypeStruct((_NENT, 2 * _FS_DIM), jnp.float32),
        compiler_params=pltpu.CompilerParams(
            dimension_semantics=("arbitrary",)),
    )(fs_t)


def _sc_body(tail_idx, pos0, pos1, ent_tab, orel_tab, rel_tab, fs_tab,
             tail_out, head_out, rel_out, orel_out,
             idx_all, bufs, pos_v, wide_v, rel_v, sem, gsem, wsem):
    c = lax.axis_index("c")
    s = lax.axis_index("s")
    wid = s * 2 + c

    pltpu.sync_copy(tail_idx.at[wid], idx_all)

    for b in range(_NBUF):
        pltpu.async_copy(fs_tab.at[idx_all.at[b]], bufs.at[b], gsem.at[b])

    pltpu.sync_copy(pos0.at[wid], pos_v)
    pltpu.async_copy(ent_tab.at[pos_v], wide_v, sem).wait()
    pltpu.sync_copy(wide_v, head_out.at[wid])
    pltpu.sync_copy(pos1.at[wid], pos_v)
    pltpu.async_copy(orel_tab.at[pos_v], wide_v, sem).wait()
    pltpu.sync_copy(wide_v, orel_out.at[wid])
    pltpu.async_copy(rel_tab.at[pos_v], rel_v, sem).wait()
    pltpu.sync_copy(rel_v, rel_out.at[wid])

    def step(i, carry):
        b = lax.rem(i, _NBUF)
        pltpu.make_async_copy(
            fs_tab.at[idx_all.at[i]], bufs.at[b], gsem.at[b]).wait()
        pltpu.async_copy(bufs.at[b], tail_out.at[wid, i], wsem.at[b])

        ip = i - 1
        nxt = ip + _NBUF

        @pl.when(jnp.logical_and(ip >= 0, nxt < _NCHUNK))
        def _():
            bp = lax.rem(ip, _NBUF)
            pltpu.make_async_copy(
                bufs.at[bp], tail_out.at[wid, ip], wsem.at[bp]).wait()
            pltpu.async_copy(
                fs_tab.at[idx_all.at[nxt]], bufs.at[bp], gsem.at[bp])

        return carry

    lax.fori_loop(0, _NCHUNK, step, 0)

    for b in range(_NBUF):
        pltpu.make_async_copy(bufs.at[b], tail_out.at[wid, 0], wsem.at[b]).wait()


@jax.jit
def _gather_all(tail_idx, pos0, pos1, ent_tab, orel_tab, rel_tab, fs_tab):
    mesh = plsc.VectorSubcoreMesh(core_axis_name="c", subcore_axis_name="s")
    f = functools.partial(
        pl.kernel,
        mesh=mesh,
        compiler_params=pltpu.CompilerParams(use_tc_tiling_on_sc=False),
        out_type=[
            jax.ShapeDtypeStruct((_NW, _NCHUNK, _SUBW, _FS_DIM), jnp.float32),
            jax.ShapeDtypeStruct((_NW, _POS_W, _ENT_DIM), jnp.float32),
            jax.ShapeDtypeStruct((_NW, _POS_W, _REL_DIM), jnp.float32),
            jax.ShapeDtypeStruct((_NW, _POS_W, _OREL_DIM), jnp.float32),
        ],
        scratch_types=[
            pltpu.VMEM((_NCHUNK, _SUBW), jnp.int32),
            pltpu.VMEM((_NBUF, _SUBW, _FS_DIM), jnp.float32),
            pltpu.VMEM((_POS_W,), jnp.int32),
            pltpu.VMEM((_POS_W, _ENT_DIM), jnp.float32),
            pltpu.VMEM((_POS_W, _REL_DIM), jnp.float32),
            pltpu.SemaphoreType.DMA,
            pltpu.SemaphoreType.DMA((_NBUF,)),
            pltpu.SemaphoreType.DMA((_NBUF,)),
        ],
    )(_sc_body)
    return f(tail_idx, pos0, pos1, ent_tab, orel_tab, rel_tab, fs_tab)


def kernel(positive, negative, entity_embedding, origin_relation_embedding,
           relation_embedding, entity_embedding_FS):
    positive = positive.astype(jnp.int32)
    negative = negative.astype(jnp.int32)
    tail_idx = jnp.concatenate([positive[:, 2:3], negative], axis=1) * 2
    tail_idx = tail_idx.reshape(_NW, _NCHUNK, _SUBW)
    pos0 = positive[:, 0].reshape(_NW, _POS_W)
    pos1 = positive[:, 1].reshape(_NW, _POS_W)
    fs_rows = _transpose_pad(entity_embedding_FS.T)
    fs_v = fs_rows.reshape(2 * _NENT, _FS_DIM)
    tail, head, rel, orel = _gather_all(
        tail_idx, pos0, pos1, entity_embedding, origin_relation_embedding,
        relation_embedding, fs_v)
    return (head.reshape(_BATCH, 1, _ENT_DIM),
            rel.reshape(_BATCH, 1, _REL_DIM),
            tail.reshape(_BATCH, _NEG + 1, _FS_DIM),
            orel.reshape(_BATCH, 1, _OREL_DIM))
